# Initial kernel scaffold; baseline (speedup 1.0000x reference)
import jax
import jax.numpy as jnp
from jax.experimental import pallas as pl


def kernel(nodes, senders, receivers, enc_W0, enc_b0, enc_W1, enc_b1, d0_W0, d0_b0, d0_W1, d0_b1, r0_W0, r0_b0, r0_W1, r0_b1, d1_W0, d1_b0, d1_W1, d1_b1, r1_W0, r1_b0, r1_W1, r1_b1, mu_W, mu_b, lv_W, lv_b, p_W0, p_b0, p_W1, p_b1):
    raise NotImplementedError("write your pallas kernel here")


# speedup vs baseline: 16.7567x; 16.7567x over previous
import functools

import jax
import jax.numpy as jnp
from jax import lax
from jax.experimental import pallas as pl
from jax.experimental.pallas import tpu as pltpu
from jax.experimental.pallas import tpu_sc as plsc

N = 10000
NP = 10240
E = 320000
D = 128
C0 = 64
C1 = 16
K = 8
LATENT = 64
BLK = 1024
NWORK = 32
EPW = E // NWORK
CH = 80
NCH = EPW // CH
ROWS_PER_SUB = NP // 16

_HIGH = lax.Precision.HIGHEST


def _dot_def(x, w):
    return jnp.dot(x.astype(jnp.bfloat16), w.astype(jnp.bfloat16),
                   preferred_element_type=jnp.float32)


def _mlp2(x, W0, b0, W1, b1):
    h = jnp.maximum(_dot_def(x, W0) + b0, 0.0)
    return _dot_def(h, W1) + b1


def _softmax(l):
    m = jnp.max(l, axis=-1, keepdims=True)
    e = jnp.exp(l - m)
    return e / jnp.sum(e, axis=-1, keepdims=True)


def _coarse_pool(nodes, asg, n_rows, nc):
    batch_idx = jnp.repeat(jnp.arange(1), jnp.array([n_rows]),
                           total_repeat_length=n_rows)
    coarse = [jax.ops.segment_sum(nodes * asg[:, kk:kk + 1], batch_idx,
                                  num_segments=1) for kk in range(nc)]
    return jnp.stack(coarse, axis=1).reshape(-1, nodes.shape[-1])


def _stage1_body(x_ref, eW0, eb0, eW1, eb1, dW0, db0, dW1, db1,
                 h_ref, asg_ref):
    h = _mlp2(x_ref[...], eW0[...], eb0[...], eW1[...], eb1[...])
    h_ref[...] = h
    asg_ref[...] = _softmax(_mlp2(h, dW0[...], db0[...], dW1[...], db1[...]))


def _stage1(nodes_p, eW0, eb0, eW1, eb1, dW0, db0, dW1, db1):
    full = lambda shape: pl.BlockSpec(shape, lambda i: (0,) * len(shape))
    return pl.pallas_call(
        _stage1_body,
        grid=(NP // BLK,),
        in_specs=[
            pl.BlockSpec((BLK, D), lambda i: (i, 0)),
            full((D, D)), full((1, D)), full((D, D)), full((1, D)),
            full((D, 32)), full((1, 32)), full((32, C0)), full((1, C0)),
        ],
        out_specs=[
            pl.BlockSpec((BLK, D), lambda i: (i, 0)),
            pl.BlockSpec((BLK, C0), lambda i: (i, 0)),
        ],
        out_shape=[
            jax.ShapeDtypeStruct((NP, D), jnp.float32),
            jax.ShapeDtypeStruct((NP, C0), jnp.float32),
        ],
        compiler_params=pltpu.CompilerParams(
            dimension_semantics=("arbitrary",)),
    )(nodes_p, eW0, eb0, eW1, eb1, dW0, db0, dW1, db1)


def _sc_body(asg_hbm, send_hbm, recv_hbm, out_hbm,
             ridx_v, sidx_v, rows_v, b_sh, sem):
    c = lax.axis_index("c")
    s = lax.axis_index("s")
    wid = s * 2 + c

    def zrow(r, _):
        for j in range(4):
            rows_v[r, pl.ds(j * 16, 16)] = jnp.zeros((16,), jnp.float32)
        return 0
    lax.fori_loop(0, CH, zrow, 0)
    for t in range(ROWS_PER_SUB // CH):
        pltpu.sync_copy(rows_v, b_sh.at[pl.ds(s * ROWS_PER_SUB + t * CH, CH)])
    plsc.subcore_barrier()

    def chunk(t, _):
        base = wid * EPW + t * CH
        pltpu.sync_copy(recv_hbm.at[pl.ds(base, CH)], ridx_v)
        pltpu.sync_copy(send_hbm.at[pl.ds(base, CH)], sidx_v)
        pltpu.async_copy(asg_hbm.at[ridx_v], rows_v, sem).wait()
        pltpu.sync_copy(rows_v, b_sh.at[sidx_v], add=True)
        return 0
    lax.fori_loop(0, NCH, chunk, 0)
    plsc.subcore_barrier()
    pltpu.sync_copy(b_sh.at[pl.ds(s * ROWS_PER_SUB, ROWS_PER_SUB)],
                    out_hbm.at[c, pl.ds(s * ROWS_PER_SUB, ROWS_PER_SUB)])


def _sc_segment(asg0_p, senders, receivers):
    mesh = plsc.VectorSubcoreMesh(core_axis_name="c", subcore_axis_name="s")
    f = functools.partial(
        pl.kernel, _sc_body, mesh=mesh,
        out_type=jax.ShapeDtypeStruct((2, NP, C0), jnp.float32),
        scratch_types=[
            pltpu.VMEM((CH,), jnp.int32),
            pltpu.VMEM((CH,), jnp.int32),
            pltpu.VMEM((CH, C0), jnp.float32),
            pltpu.VMEM_SHARED((NP, C0), jnp.float32),
            pltpu.SemaphoreType.DMA,
        ],
        compiler_params=pltpu.CompilerParams(use_tc_tiling_on_sc=False),
    )()
    return f(asg0_p, senders, receivers)


def _topk_onehot_sum(a, n, k):
    cols = lax.broadcasted_iota(jnp.int32, (n, n), 1)
    P = jnp.zeros((n, n), jnp.float32)
    for _ in range(k):
        m = jnp.max(a, axis=1, keepdims=True)
        idx = jnp.min(jnp.where(a == m, cols, n), axis=1, keepdims=True)
        oh = (cols == idx).astype(jnp.float32)
        P = P + oh
        a = jnp.where(oh > 0.0, -1e30, a)
    return P


def _topk_vals(a, n, k):
    cols = lax.broadcasted_iota(jnp.int32, (n, n), 1)
    vals = []
    for _ in range(k):
        m = jnp.max(a, axis=1, keepdims=True)
        idx = jnp.min(jnp.where(a == m, cols, n), axis=1, keepdims=True)
        vals.append(m)
        a = jnp.where(cols == idx, -1e30, a)
    return jnp.concatenate(vals, axis=1)


def _stage2a_body(asg_ref, b0_ref, b1_ref, coarse_ref,
                  r0W0, r0b0, r0W1, r0b1, d1W0, d1b0, d1W1, d1b1,
                  P_ref, nodes1_ref, asg1_ref, adj_scr):
    i = pl.program_id(0)
    b = b0_ref[...] + b1_ref[...]
    contrib = lax.dot_general(asg_ref[...], b, (((0,), (0,)), ((), ())),
                              precision=_HIGH)

    @pl.when(i == 0)
    def _():
        adj_scr[...] = contrib

    @pl.when(i > 0)
    def _():
        adj_scr[...] += contrib

    @pl.when(i == pl.num_programs(0) - 1)
    def _():
        P_ref[...] = _topk_onehot_sum(adj_scr[...], C0, K)
        nodes1 = _mlp2(coarse_ref[...], r0W0[...], r0b0[...],
                       r0W1[...], r0b1[...])
        nodes1_ref[...] = nodes1
        asg1_ref[...] = _softmax(_mlp2(nodes1, d1W0[...], d1b0[...],
                                       d1W1[...], d1b1[...]))


def _stage2a(asg0_p, b0, b1, coarse0, r0W0, r0b0, r0W1, r0b1,
             d1W0, d1b0, d1W1, d1b1):
    full = lambda shape: pl.BlockSpec(shape, lambda i: (0,) * len(shape))
    return pl.pallas_call(
        _stage2a_body,
        grid=(NP // BLK,),
        in_specs=[
            pl.BlockSpec((BLK, C0), lambda i: (i, 0)),
            pl.BlockSpec((BLK, C0), lambda i: (i, 0)),
            pl.BlockSpec((BLK, C0), lambda i: (i, 0)),
            full((C0, D)),
            full((D, D)), full((1, D)), full((D, D)), full((1, D)),
            full((D, 32)), full((1, 32)), full((32, C1)), full((1, C1)),
        ],
        out_specs=[full((C0, C0)), full((C0, D)), full((C0, C1))],
        out_shape=[
            jax.ShapeDtypeStruct((C0, C0), jnp.float32),
            jax.ShapeDtypeStruct((C0, D), jnp.float32),
            jax.ShapeDtypeStruct((C0, C1), jnp.float32),
        ],
        scratch_shapes=[pltpu.VMEM((C0, C0), jnp.float32)],
        compiler_params=pltpu.CompilerParams(
            dimension_semantics=("arbitrary",)),
    )(asg0_p, b0, b1, coarse0, r0W0, r0b0, r0W1, r0b1,
      d1W0, d1b0, d1W1, d1b1)


def _stage2b_body(P_ref, asg1_ref, coarse1_ref,
                  r1W0, r1b0, r1W1, r1b1, muW, mub, lvW, lvb,
                  pW0, pb0, eps_ref,
                  mu_ref, lv_ref, h1_ref, nodes2_ref, ew2_ref):
    asg1 = asg1_ref[...]
    Ms = jnp.dot(P_ref[...], asg1, precision=_HIGH)
    adj1 = lax.dot_general(asg1, Ms, (((0,), (0,)), ((), ())),
                           precision=_HIGH)
    ew2_ref[...] = _topk_vals(adj1, C1, K)
    nodes2 = _mlp2(coarse1_ref[...], r1W0[...], r1b0[...],
                   r1W1[...], r1b1[...])
    nodes2_ref[...] = nodes2
    macro = nodes2[0:1, :]
    for r in range(1, C1):
        macro = macro + nodes2[r:r + 1, :]
    macro = macro * (1.0 / C1)
    mu = _dot_def(macro, muW[...]) + mub[...]
    lv = _dot_def(macro, lvW[...]) + lvb[...]
    mu_ref[...] = mu
    lv_ref[...] = lv
    z = mu + eps_ref[...] * jnp.exp(0.5 * lv)
    h1_ref[...] = jnp.maximum(_dot_def(z, pW0[...]) + pb0[...], 0.0)


def _stage2b(P, asg1, coarse1, r1W0, r1b0, r1W1, r1b1,
             muW, mub, lvW, lvb, pW0, pb0, eps):
    full = lambda shape: pl.BlockSpec(shape, lambda *_: (0,) * len(shape))
    return pl.pallas_call(
        _stage2b_body,
        in_specs=[
            full((C0, C0)), full((C0, C1)), full((C1, D)),
            full((D, D)), full((1, D)), full((D, D)), full((1, D)),
            full((D, LATENT)), full((1, LATENT)),
            full((D, LATENT)), full((1, LATENT)),
            full((LATENT, 32)), full((1, 32)),
            full((1, LATENT)),
        ],
        out_specs=[
            full((1, LATENT)), full((1, LATENT)), full((1, 32)),
            full((C1, D)), full((C1, K)),
        ],
        out_shape=[
            jax.ShapeDtypeStruct((1, LATENT), jnp.float32),
            jax.ShapeDtypeStruct((1, LATENT), jnp.float32),
            jax.ShapeDtypeStruct((1, 32), jnp.float32),
            jax.ShapeDtypeStruct((C1, D), jnp.float32),
            jax.ShapeDtypeStruct((C1, K), jnp.float32),
        ],
    )(P, asg1, coarse1, r1W0, r1b0, r1W1, r1b1,
      muW, mub, lvW, lvb, pW0, pb0, eps)


def kernel(nodes, senders, receivers, enc_W0, enc_b0, enc_W1, enc_b1,
           d0_W0, d0_b0, d0_W1, d0_b1, r0_W0, r0_b0, r0_W1, r0_b1,
           d1_W0, d1_b0, d1_W1, d1_b1, r1_W0, r1_b0, r1_W1, r1_b1,
           mu_W, mu_b, lv_W, lv_b, p_W0, p_b0, p_W1, p_b1):
    r2 = lambda v: v.reshape(1, -1)
    nodes_p = jnp.pad(nodes, ((0, NP - N), (0, 0)))
    eps = jax.random.normal(jax.random.PRNGKey(0), (1, LATENT))

    h_p, asg0_p = _stage1(nodes_p, enc_W0, r2(enc_b0), enc_W1, r2(enc_b1),
                          d0_W0, r2(d0_b0), d0_W1, r2(d0_b1))
    coarse0 = _coarse_pool(h_p[:N], asg0_p[:N], N, C0)
    bparts = _sc_segment(asg0_p, senders, receivers)
    P, nodes1, asg1 = _stage2a(asg0_p, bparts[0], bparts[1], coarse0,
                               r0_W0, r2(r0_b0), r0_W1, r2(r0_b1),
                               d1_W0, r2(d1_b0), d1_W1, r2(d1_b1))
    coarse1 = _coarse_pool(nodes1, asg1, C0, C1)
    mu, lv, h1, nodes2, ew2k = _stage2b(
        P, asg1, coarse1,
        r1_W0, r2(r1_b0), r1_W1, r2(r1_b1),
        mu_W, r2(mu_b), lv_W, r2(lv_b),
        p_W0, r2(p_b0), eps)
    pred = h1 @ p_W1 + p_b1
    return (mu, lv, pred, asg0_p[:N], asg1, nodes2, ew2k.reshape(C1 * K, 1))


# speedup vs baseline: 17.3598x; 1.0360x over previous
import functools

import jax
import jax.numpy as jnp
from jax import lax
from jax.experimental import pallas as pl
from jax.experimental.pallas import tpu as pltpu
from jax.experimental.pallas import tpu_sc as plsc

N = 10000
NP = 10240
E = 320000
D = 128
C0 = 64
C1 = 16
K = 8
LATENT = 64
BLK = 1024
NWORK = 32
EPW = E // NWORK
CH = 80
NCH = EPW // CH
ROWS_PER_SUB = NP // 16

_HIGH = lax.Precision.HIGHEST


def _dot_def(x, w):
    return jnp.dot(x.astype(jnp.bfloat16), w.astype(jnp.bfloat16),
                   preferred_element_type=jnp.float32)


def _mlp2(x, W0, b0, W1, b1):
    h = jnp.maximum(_dot_def(x, W0) + b0, 0.0)
    return _dot_def(h, W1) + b1


def _softmax(l):
    m = jnp.max(l, axis=-1, keepdims=True)
    e = jnp.exp(l - m)
    return e / jnp.sum(e, axis=-1, keepdims=True)


def _coarse_pool(nodes, asg, n_rows, nc):
    batch_idx = jnp.repeat(jnp.arange(1), jnp.array([n_rows]),
                           total_repeat_length=n_rows)
    coarse = [jax.ops.segment_sum(nodes * asg[:, kk:kk + 1], batch_idx,
                                  num_segments=1) for kk in range(nc)]
    return jnp.stack(coarse, axis=1).reshape(-1, nodes.shape[-1])


def _stage1_body(x_ref, eW0, eb0, eW1, eb1, dW0, db0, dW1, db1,
                 h_ref, asg_ref):
    h = _mlp2(x_ref[...], eW0[...], eb0[...], eW1[...], eb1[...])
    h_ref[...] = h
    asg_ref[...] = _softmax(_mlp2(h, dW0[...], db0[...], dW1[...], db1[...]))


def _stage1(nodes_p, eW0, eb0, eW1, eb1, dW0, db0, dW1, db1):
    full = lambda shape: pl.BlockSpec(shape, lambda i: (0,) * len(shape))
    return pl.pallas_call(
        _stage1_body,
        grid=(NP // BLK,),
        in_specs=[
            pl.BlockSpec((BLK, D), lambda i: (i, 0)),
            full((D, D)), full((1, D)), full((D, D)), full((1, D)),
            full((D, 32)), full((1, 32)), full((32, C0)), full((1, C0)),
        ],
        out_specs=[
            pl.BlockSpec((BLK, D), lambda i: (i, 0)),
            pl.BlockSpec((BLK, C0), lambda i: (i, 0)),
        ],
        out_shape=[
            jax.ShapeDtypeStruct((NP, D), jnp.float32),
            jax.ShapeDtypeStruct((NP, C0), jnp.float32),
        ],
        compiler_params=pltpu.CompilerParams(
            dimension_semantics=("arbitrary",)),
    )(nodes_p, eW0, eb0, eW1, eb1, dW0, db0, dW1, db1)


def _sc_body(asg_hbm, send_hbm, recv_hbm, out_hbm,
             ridx_v, sidx_v, rows0_v, rows1_v, b_sh, sem0, sem1):
    c = lax.axis_index("c")
    s = lax.axis_index("s")
    wid = s * 2 + c

    def zrow(r, _):
        for j in range(4):
            rows0_v[r, pl.ds(j * 16, 16)] = jnp.zeros((16,), jnp.float32)
        return 0
    lax.fori_loop(0, CH, zrow, 0)
    for t in range(ROWS_PER_SUB // CH):
        pltpu.sync_copy(rows0_v, b_sh.at[pl.ds(s * ROWS_PER_SUB + t * CH, CH)])
    plsc.subcore_barrier()

    pltpu.sync_copy(recv_hbm.at[wid], ridx_v)
    pltpu.sync_copy(send_hbm.at[wid], sidx_v)

    def pair(i, _):
        t0 = 2 * i
        t1 = 2 * i + 1
        cp0 = pltpu.async_copy(asg_hbm.at[ridx_v.at[t0]], rows0_v, sem0)
        cp1 = pltpu.async_copy(asg_hbm.at[ridx_v.at[t1]], rows1_v, sem1)
        cp0.wait()
        pltpu.sync_copy(rows0_v, b_sh.at[sidx_v.at[t0]], add=True)
        cp1.wait()
        pltpu.sync_copy(rows1_v, b_sh.at[sidx_v.at[t1]], add=True)
        return 0
    lax.fori_loop(0, NCH // 2, pair, 0)
    for t in range(NCH - 2 * (NCH // 2), 0, -1):
        tt = NCH - t
        pltpu.async_copy(asg_hbm.at[ridx_v.at[tt]], rows0_v, sem0).wait()
        pltpu.sync_copy(rows0_v, b_sh.at[sidx_v.at[tt]], add=True)
    plsc.subcore_barrier()
    pltpu.sync_copy(b_sh.at[pl.ds(s * ROWS_PER_SUB, ROWS_PER_SUB)],
                    out_hbm.at[c, pl.ds(s * ROWS_PER_SUB, ROWS_PER_SUB)])


def _sc_segment(asg0_p, senders, receivers):
    mesh = plsc.VectorSubcoreMesh(core_axis_name="c", subcore_axis_name="s")
    f = functools.partial(
        pl.kernel, _sc_body, mesh=mesh,
        out_type=jax.ShapeDtypeStruct((2, NP, C0), jnp.float32),
        scratch_types=[
            pltpu.VMEM((NCH, CH), jnp.int32),
            pltpu.VMEM((NCH, CH), jnp.int32),
            pltpu.VMEM((CH, C0), jnp.float32),
            pltpu.VMEM((CH, C0), jnp.float32),
            pltpu.VMEM_SHARED((NP, C0), jnp.float32),
            pltpu.SemaphoreType.DMA,
            pltpu.SemaphoreType.DMA,
        ],
        compiler_params=pltpu.CompilerParams(use_tc_tiling_on_sc=False),
    )()
    return f(asg0_p, senders.reshape(NWORK, NCH, CH),
             receivers.reshape(NWORK, NCH, CH))


def _topk_onehot_sum(a, n, k):
    cols = lax.broadcasted_iota(jnp.int32, (n, n), 1)
    P = jnp.zeros((n, n), jnp.float32)
    for _ in range(k):
        m = jnp.max(a, axis=1, keepdims=True)
        idx = jnp.min(jnp.where(a == m, cols, n), axis=1, keepdims=True)
        oh = (cols == idx).astype(jnp.float32)
        P = P + oh
        a = jnp.where(oh > 0.0, -1e30, a)
    return P


def _topk_vals(a, n, k):
    cols = lax.broadcasted_iota(jnp.int32, (n, n), 1)
    vals = []
    for _ in range(k):
        m = jnp.max(a, axis=1, keepdims=True)
        idx = jnp.min(jnp.where(a == m, cols, n), axis=1, keepdims=True)
        vals.append(m)
        a = jnp.where(cols == idx, -1e30, a)
    return jnp.concatenate(vals, axis=1)


def _stage2a_body(asg_ref, b0_ref, b1_ref, coarse_ref,
                  r0W0, r0b0, r0W1, r0b1, d1W0, d1b0, d1W1, d1b1,
                  P_ref, nodes1_ref, asg1_ref, adj_scr):
    i = pl.program_id(0)
    b = b0_ref[...] + b1_ref[...]
    contrib = lax.dot_general(asg_ref[...], b, (((0,), (0,)), ((), ())),
                              precision=_HIGH)

    @pl.when(i == 0)
    def _():
        adj_scr[...] = contrib

    @pl.when(i > 0)
    def _():
        adj_scr[...] += contrib

    @pl.when(i == pl.num_programs(0) - 1)
    def _():
        P_ref[...] = _topk_onehot_sum(adj_scr[...], C0, K)
        nodes1 = _mlp2(coarse_ref[...], r0W0[...], r0b0[...],
                       r0W1[...], r0b1[...])
        nodes1_ref[...] = nodes1
        asg1_ref[...] = _softmax(_mlp2(nodes1, d1W0[...], d1b0[...],
                                       d1W1[...], d1b1[...]))


def _stage2a(asg0_p, b0, b1, coarse0, r0W0, r0b0, r0W1, r0b1,
             d1W0, d1b0, d1W1, d1b1):
    full = lambda shape: pl.BlockSpec(shape, lambda i: (0,) * len(shape))
    return pl.pallas_call(
        _stage2a_body,
        grid=(NP // BLK,),
        in_specs=[
            pl.BlockSpec((BLK, C0), lambda i: (i, 0)),
            pl.BlockSpec((BLK, C0), lambda i: (i, 0)),
            pl.BlockSpec((BLK, C0), lambda i: (i, 0)),
            full((C0, D)),
            full((D, D)), full((1, D)), full((D, D)), full((1, D)),
            full((D, 32)), full((1, 32)), full((32, C1)), full((1, C1)),
        ],
        out_specs=[full((C0, C0)), full((C0, D)), full((C0, C1))],
        out_shape=[
            jax.ShapeDtypeStruct((C0, C0), jnp.float32),
            jax.ShapeDtypeStruct((C0, D), jnp.float32),
            jax.ShapeDtypeStruct((C0, C1), jnp.float32),
        ],
        scratch_shapes=[pltpu.VMEM((C0, C0), jnp.float32)],
        compiler_params=pltpu.CompilerParams(
            dimension_semantics=("arbitrary",)),
    )(asg0_p, b0, b1, coarse0, r0W0, r0b0, r0W1, r0b1,
      d1W0, d1b0, d1W1, d1b1)


def _stage2b_body(P_ref, asg1_ref, coarse1_ref,
                  r1W0, r1b0, r1W1, r1b1, muW, mub, lvW, lvb,
                  pW0, pb0, eps_ref,
                  mu_ref, lv_ref, h1_ref, nodes2_ref, ew2_ref):
    asg1 = asg1_ref[...]
    Ms = jnp.dot(P_ref[...], asg1, precision=_HIGH)
    adj1 = lax.dot_general(asg1, Ms, (((0,), (0,)), ((), ())),
                           precision=_HIGH)
    ew2_ref[...] = _topk_vals(adj1, C1, K)
    nodes2 = _mlp2(coarse1_ref[...], r1W0[...], r1b0[...],
                   r1W1[...], r1b1[...])
    nodes2_ref[...] = nodes2
    macro = nodes2[0:1, :]
    for r in range(1, C1):
        macro = macro + nodes2[r:r + 1, :]
    macro = macro * (1.0 / C1)
    mu = _dot_def(macro, muW[...]) + mub[...]
    lv = _dot_def(macro, lvW[...]) + lvb[...]
    mu_ref[...] = mu
    lv_ref[...] = lv
    z = mu + eps_ref[...] * jnp.exp(0.5 * lv)
    h1_ref[...] = jnp.maximum(_dot_def(z, pW0[...]) + pb0[...], 0.0)


def _stage2b(P, asg1, coarse1, r1W0, r1b0, r1W1, r1b1,
             muW, mub, lvW, lvb, pW0, pb0, eps):
    full = lambda shape: pl.BlockSpec(shape, lambda *_: (0,) * len(shape))
    return pl.pallas_call(
        _stage2b_body,
        in_specs=[
            full((C0, C0)), full((C0, C1)), full((C1, D)),
            full((D, D)), full((1, D)), full((D, D)), full((1, D)),
            full((D, LATENT)), full((1, LATENT)),
            full((D, LATENT)), full((1, LATENT)),
            full((LATENT, 32)), full((1, 32)),
            full((1, LATENT)),
        ],
        out_specs=[
            full((1, LATENT)), full((1, LATENT)), full((1, 32)),
            full((C1, D)), full((C1, K)),
        ],
        out_shape=[
            jax.ShapeDtypeStruct((1, LATENT), jnp.float32),
            jax.ShapeDtypeStruct((1, LATENT), jnp.float32),
            jax.ShapeDtypeStruct((1, 32), jnp.float32),
            jax.ShapeDtypeStruct((C1, D), jnp.float32),
            jax.ShapeDtypeStruct((C1, K), jnp.float32),
        ],
    )(P, asg1, coarse1, r1W0, r1b0, r1W1, r1b1,
      muW, mub, lvW, lvb, pW0, pb0, eps)


def kernel(nodes, senders, receivers, enc_W0, enc_b0, enc_W1, enc_b1,
           d0_W0, d0_b0, d0_W1, d0_b1, r0_W0, r0_b0, r0_W1, r0_b1,
           d1_W0, d1_b0, d1_W1, d1_b1, r1_W0, r1_b0, r1_W1, r1_b1,
           mu_W, mu_b, lv_W, lv_b, p_W0, p_b0, p_W1, p_b1):
    r2 = lambda v: v.reshape(1, -1)
    nodes_p = jnp.pad(nodes, ((0, NP - N), (0, 0)))
    eps = jax.random.normal(jax.random.PRNGKey(0), (1, LATENT))

    h_p, asg0_p = _stage1(nodes_p, enc_W0, r2(enc_b0), enc_W1, r2(enc_b1),
                          d0_W0, r2(d0_b0), d0_W1, r2(d0_b1))
    coarse0 = _coarse_pool(h_p[:N], asg0_p[:N], N, C0)
    bparts = _sc_segment(asg0_p, senders, receivers)
    P, nodes1, asg1 = _stage2a(asg0_p, bparts[0], bparts[1], coarse0,
                               r0_W0, r2(r0_b0), r0_W1, r2(r0_b1),
                               d1_W0, r2(d1_b0), d1_W1, r2(d1_b1))
    coarse1 = _coarse_pool(nodes1, asg1, C0, C1)
    mu, lv, h1, nodes2, ew2k = _stage2b(
        P, asg1, coarse1,
        r1_W0, r2(r1_b0), r1_W1, r2(r1_b1),
        mu_W, r2(mu_b), lv_W, r2(lv_b),
        p_W0, r2(p_b0), eps)
    pred = h1 @ p_W1 + p_b1
    return (mu, lv, pred, asg0_p[:N], asg1, nodes2, ew2k.reshape(C1 * K, 1))
